# baseline (speedup 1.0000x reference)
import functools

import jax
import jax.numpy as jnp
from jax import lax
from jax.experimental import pallas as pl
from jax.experimental.pallas import tpu as pltpu
from jax.experimental.pallas import tpu_sc as plsc

N_NODES = 10000
E_EDGES = 160000
D_IN, D_HID, D_OUT = 128, 128, 40
W0_AUG = 136
W1_AUG = 48

CHUNK = 100
GROUP = 10
NC, NS = 2, 16
NW = NC * NS
N_PAD = 10240
ROWS_PER_TILE = N_PAD // NS
BLK = 1000


def _sc_segment_sum(tables, edges, zrows, width, chunk, group, nbuf):
    n_chunks = E_EDGES // chunk
    cpt = n_chunks // NW
    n_groups = cpt // group
    blocks = group // nbuf
    assert cpt % group == 0 and group % nbuf == 0

    mesh = plsc.VectorSubcoreMesh(core_axis_name="c", subcore_axis_name="s")

    def body(t0, t1, t2, e0, e1, e2, zsrc, part, sidx, didx, *rest):
        rbufs = rest[0:nbuf]
        acc = rest[nbuf]
        gsems = rest[nbuf + 1:2 * nbuf + 1]
        ssems = rest[2 * nbuf + 1:3 * nbuf + 1]
        c = lax.axis_index("c")
        s = lax.axis_index("s")
        w = s * NC + c

        def wait_gather(tab, i):
            pltpu.make_async_copy(
                tab.at[pl.ds(0, chunk)], rbufs[i], gsems[i]).wait()

        def wait_scatter(i):
            pltpu.make_async_copy(
                rbufs[i], acc.at[pl.ds(0, chunk)], ssems[i]).wait()

        for r in range(3):
            tab = (t0, t1, t2)[r]
            edge = (e0, e1, e2)[r]

            pltpu.sync_copy(zsrc, acc.at[pl.ds(s * ROWS_PER_TILE,
                                               ROWS_PER_TILE)])
            plsc.subcore_barrier()

            def group_body(g, carry):
                base = w * cpt + g * group
                pltpu.sync_copy(edge.at[0, pl.ds(base, group)], sidx)
                pltpu.sync_copy(edge.at[1, pl.ds(base, group)], didx)
                for i in range(nbuf - 1):
                    @pl.when(g > 0)
                    def _(i=i):
                        wait_scatter(i)
                    pltpu.async_copy(tab.at[sidx.at[i]], rbufs[i], gsems[i])

                def block_body(blk, carry2):
                    for i in range(nbuf):
                        j = blk * nbuf + i
                        wait_gather(tab, i)
                        pltpu.async_copy(
                            rbufs[i], acc.at[didx.at[j]], ssems[i], add=True)
                        prev = (i - 1) % nbuf

                        def issue(blk=blk, i=i, prev=prev, j=j):
                            pltpu.async_copy(
                                tab.at[sidx.at[j + nbuf - 1]],
                                rbufs[prev], gsems[prev])

                        if i == 0:
                            @pl.when(jnp.logical_or(g > 0, blk > 0))
                            def _():
                                wait_scatter(prev)
                            issue()
                        else:
                            @pl.when(blk < blocks - 1)
                            def _():
                                wait_scatter(prev)
                                issue()
                    return carry2
                lax.fori_loop(0, blocks, block_body, 0)
                return carry
            lax.fori_loop(0, n_groups, group_body, 0)
            for i in range(nbuf):
                wait_scatter(i)
            plsc.subcore_barrier()

            pltpu.sync_copy(
                acc.at[pl.ds(s * ROWS_PER_TILE, ROWS_PER_TILE)],
                part.at[r, c, pl.ds(s * ROWS_PER_TILE, ROWS_PER_TILE)])
            plsc.subcore_barrier()

    fn = pl.kernel(
        body,
        out_type=jax.ShapeDtypeStruct((3, 2, N_PAD, width), jnp.float32),
        mesh=mesh,
        scratch_types=(
            [pltpu.VMEM((group, chunk), jnp.int32)] * 2
            + [pltpu.VMEM((chunk, width), jnp.float32)] * nbuf
            + [pltpu.VMEM_SHARED((N_PAD, width), jnp.float32)]
            + [pltpu.SemaphoreType.DMA] * (2 * nbuf)
        ),
        compiler_params=pltpu.CompilerParams(use_tc_tiling_on_sc=False),
    )
    return fn(*tables, *edges, zrows)


def _tc_combine_matmul(part, W0s, b0s, W1s, b1s, d_feat, width_out):
    width_in = part.shape[-1]

    def body(p_ref, w00, w01, w02, b00, b01, b02,
             w10, w11, w12, b10, b11, b12, o0, o1, o2):
        pv = p_ref[...]
        h = None
        for w0r, b0r, r in ((w00, b00, 0), (w01, b01, 1), (w02, b02, 2)):
            srel = pv[r, 0] + pv[r, 1]
            cntcol = srel[:, d_feat:d_feat + 1]
            aggf = srel[:, :d_feat] / jnp.maximum(cntcol, 1.0)
            m = (jnp.dot(aggf, w0r[...], preferred_element_type=jnp.float32)
                 + b0r[...] * (cntcol > 0.0).astype(jnp.float32))
            h = m if h is None else h + m
        h = jnp.maximum(h, 0.0)
        for wr, br, orf in ((w10, b10, o0), (w11, b11, o1), (w12, b12, o2)):
            orf[...] = (jnp.dot(h, wr[...],
                                preferred_element_type=jnp.float32)
                        + br[...])

    return pl.pallas_call(
        body,
        grid=(N_NODES // BLK,),
        in_specs=[pl.BlockSpec((3, 2, BLK, width_in), lambda i: (0, 0, i, 0))]
        + [pl.BlockSpec((d_feat, d_feat), lambda i: (0, 0))] * 3
        + [pl.BlockSpec((1, d_feat), lambda i: (0, 0))] * 3
        + [pl.BlockSpec((d_feat, width_out), lambda i: (0, 0))] * 3
        + [pl.BlockSpec((1, width_out), lambda i: (0, 0))] * 3,
        out_specs=[pl.BlockSpec((BLK, width_out), lambda i: (i, 0))] * 3,
        out_shape=[jax.ShapeDtypeStruct((N_NODES, width_out), jnp.float32)] * 3,
    )(part, *W0s, *b0s, *W1s, *b1s)


def _tc_combine_final(part, d_feat):
    width_in = part.shape[-1]

    def body(p_ref, o_ref):
        pv = p_ref[...]
        h = None
        for r in range(3):
            srel = pv[r, 0] + pv[r, 1]
            cnt = jnp.maximum(srel[:, d_feat:d_feat + 1], 1.0)
            m = srel[:, :d_feat] / cnt
            h = m if h is None else h + m
        o_ref[...] = h

    return pl.pallas_call(
        body,
        grid=(N_NODES // BLK,),
        in_specs=[pl.BlockSpec((3, 2, BLK, width_in), lambda i: (0, 0, i, 0))],
        out_specs=pl.BlockSpec((BLK, d_feat), lambda i: (i, 0)),
        out_shape=jax.ShapeDtypeStruct((N_NODES, d_feat), jnp.float32),
    )(part)


def _augment(W, b, width):
    d_in, d = W.shape
    Waug = jnp.pad(W, ((0, 0), (0, width - d)))
    baug = jnp.concatenate(
        [b, jnp.ones((1,), b.dtype), jnp.zeros((width - d - 1,), b.dtype)]
    ).reshape(1, width)
    return Waug, baug


def kernel(feat, edge_index_r0, edge_index_r1, edge_index_r2,
           W0_r0, b0_r0, W1_r0, b1_r0,
           W0_r1, b0_r1, W1_r1, b1_r1,
           W0_r2, b0_r2, W1_r2, b1_r2):
    edges_raw = (edge_index_r0, edge_index_r1, edge_index_r2)
    edges0 = [e.reshape(2, E_EDGES // 50, 50) for e in edges_raw]
    edges1 = [e.reshape(2, E_EDGES // 125, 125) for e in edges_raw]

    W0s = [W0_r0, W0_r1, W0_r2]
    b0s = [b.reshape(1, D_HID) for b in (b0_r0, b0_r1, b0_r2)]
    W1s, b1s = zip(*(_augment(W, b, W1_AUG)
                     for W, b in ((W1_r0, b1_r0), (W1_r1, b1_r1), (W1_r2, b1_r2))))

    feataug = jnp.concatenate(
        [feat,
         jnp.ones((N_NODES, 1), feat.dtype),
         jnp.zeros((N_NODES, W0_AUG - D_IN - 1), feat.dtype)], axis=1)
    z0 = jnp.zeros((ROWS_PER_TILE, W0_AUG), jnp.float32)
    P0 = _sc_segment_sum([feataug] * 3, edges0, z0, W0_AUG,
                         chunk=50, group=20, nbuf=4)
    T1 = _tc_combine_matmul(P0, W0s, b0s, list(W1s), list(b1s),
                            D_HID, W1_AUG)
    z1 = jnp.zeros((ROWS_PER_TILE, W1_AUG), jnp.float32)
    P1 = _sc_segment_sum(list(T1), edges1, z1, W1_AUG,
                         chunk=125, group=40, nbuf=4)
    return _tc_combine_final(P1, D_OUT)


# speedup vs baseline: 1.0309x; 1.0309x over previous
import functools

import jax
import jax.numpy as jnp
from jax import lax
from jax.experimental import pallas as pl
from jax.experimental.pallas import tpu as pltpu
from jax.experimental.pallas import tpu_sc as plsc

N_NODES = 10000
E_EDGES = 160000
D_IN, D_HID, D_OUT = 128, 128, 40
W0_AUG = 144
W1_AUG = 48

CHUNK = 100
GROUP = 10
NC, NS = 2, 16
NW = NC * NS
N_PAD = 10240
ROWS_PER_TILE = N_PAD // NS
BLK = 1000


def _sc_segment_sum(tables, edges, zrows, width, chunk, group, nbuf):
    n_chunks = E_EDGES // chunk
    cpt = n_chunks // NW
    n_groups = cpt // group
    blocks = group // nbuf
    assert cpt % group == 0 and group % nbuf == 0

    mesh = plsc.VectorSubcoreMesh(core_axis_name="c", subcore_axis_name="s")

    def body(t0, t1, t2, e0, e1, e2, zsrc, part, sidx, didx, *rest):
        rbufs = rest[0:nbuf]
        acc = rest[nbuf]
        gsems = rest[nbuf + 1:2 * nbuf + 1]
        ssems = rest[2 * nbuf + 1:3 * nbuf + 1]
        c = lax.axis_index("c")
        s = lax.axis_index("s")
        w = s * NC + c

        def wait_gather(tab, i):
            pltpu.make_async_copy(
                tab.at[pl.ds(0, chunk)], rbufs[i], gsems[i]).wait()

        def wait_scatter(i):
            pltpu.make_async_copy(
                rbufs[i], acc.at[pl.ds(0, chunk)], ssems[i]).wait()

        for r in range(3):
            tab = (t0, t1, t2)[r]
            edge = (e0, e1, e2)[r]

            pltpu.sync_copy(zsrc, acc.at[pl.ds(s * ROWS_PER_TILE,
                                               ROWS_PER_TILE)])
            plsc.subcore_barrier()

            def group_body(g, carry):
                base = w * cpt + g * group
                pltpu.sync_copy(edge.at[0, pl.ds(base, group)], sidx)
                pltpu.sync_copy(edge.at[1, pl.ds(base, group)], didx)
                for i in range(nbuf - 1):
                    @pl.when(g > 0)
                    def _(i=i):
                        wait_scatter(i)
                    pltpu.async_copy(tab.at[sidx.at[i]], rbufs[i], gsems[i])

                def block_body(blk, carry2):
                    for i in range(nbuf):
                        j = blk * nbuf + i
                        wait_gather(tab, i)
                        pltpu.async_copy(
                            rbufs[i], acc.at[didx.at[j]], ssems[i], add=True)
                        prev = (i - 1) % nbuf

                        def issue(blk=blk, i=i, prev=prev, j=j):
                            pltpu.async_copy(
                                tab.at[sidx.at[j + nbuf - 1]],
                                rbufs[prev], gsems[prev])

                        if i == 0:
                            @pl.when(jnp.logical_or(g > 0, blk > 0))
                            def _():
                                wait_scatter(prev)
                            issue()
                        else:
                            @pl.when(blk < blocks - 1)
                            def _():
                                wait_scatter(prev)
                                issue()
                    return carry2
                lax.fori_loop(0, blocks, block_body, 0)
                return carry
            lax.fori_loop(0, n_groups, group_body, 0)
            for i in range(nbuf):
                wait_scatter(i)
            plsc.subcore_barrier()

            pltpu.sync_copy(
                acc.at[pl.ds(s * ROWS_PER_TILE, ROWS_PER_TILE)],
                part.at[r, c, pl.ds(s * ROWS_PER_TILE, ROWS_PER_TILE)])
            plsc.subcore_barrier()

    fn = pl.kernel(
        body,
        out_type=jax.ShapeDtypeStruct((3, 2, N_PAD, width), jnp.float32),
        mesh=mesh,
        scratch_types=(
            [pltpu.VMEM((group, chunk), jnp.int32)] * 2
            + [pltpu.VMEM((chunk, width), jnp.float32)] * nbuf
            + [pltpu.VMEM_SHARED((N_PAD, width), jnp.float32)]
            + [pltpu.SemaphoreType.DMA] * (2 * nbuf)
        ),
        compiler_params=pltpu.CompilerParams(use_tc_tiling_on_sc=False),
    )
    return fn(*tables, *edges, zrows)


def _tc_combine_matmul(part, W0s, b0s, W1s, b1s, d_feat, width_out):
    width_in = part.shape[-1]

    def body(p_ref, w00, w01, w02, b00, b01, b02,
             w10, w11, w12, b10, b11, b12, o0, o1, o2):
        pv = p_ref[...]
        h = None
        for w0r, b0r, r in ((w00, b00, 0), (w01, b01, 1), (w02, b02, 2)):
            srel = pv[r, 0] + pv[r, 1]
            cntcol = srel[:, d_feat:d_feat + 1]
            aggf = srel[:, :d_feat] / jnp.maximum(cntcol, 1.0)
            m = (jnp.dot(aggf, w0r[...], preferred_element_type=jnp.float32)
                 + b0r[...] * (cntcol > 0.0).astype(jnp.float32))
            h = m if h is None else h + m
        h = jnp.maximum(h, 0.0)
        for wr, br, orf in ((w10, b10, o0), (w11, b11, o1), (w12, b12, o2)):
            orf[...] = (jnp.dot(h, wr[...],
                                preferred_element_type=jnp.float32)
                        + br[...])

    return pl.pallas_call(
        body,
        grid=(N_NODES // BLK,),
        in_specs=[pl.BlockSpec((3, 2, BLK, width_in), lambda i: (0, 0, i, 0))]
        + [pl.BlockSpec((d_feat, d_feat), lambda i: (0, 0))] * 3
        + [pl.BlockSpec((1, d_feat), lambda i: (0, 0))] * 3
        + [pl.BlockSpec((d_feat, width_out), lambda i: (0, 0))] * 3
        + [pl.BlockSpec((1, width_out), lambda i: (0, 0))] * 3,
        out_specs=[pl.BlockSpec((BLK, width_out), lambda i: (i, 0))] * 3,
        out_shape=[jax.ShapeDtypeStruct((N_NODES, width_out), jnp.float32)] * 3,
    )(part, *W0s, *b0s, *W1s, *b1s)


def _tc_combine_final(part, d_feat):
    width_in = part.shape[-1]

    def body(p_ref, o_ref):
        pv = p_ref[...]
        h = None
        for r in range(3):
            srel = pv[r, 0] + pv[r, 1]
            cnt = jnp.maximum(srel[:, d_feat:d_feat + 1], 1.0)
            m = srel[:, :d_feat] / cnt
            h = m if h is None else h + m
        o_ref[...] = h

    return pl.pallas_call(
        body,
        grid=(N_NODES // BLK,),
        in_specs=[pl.BlockSpec((3, 2, BLK, width_in), lambda i: (0, 0, i, 0))],
        out_specs=pl.BlockSpec((BLK, d_feat), lambda i: (i, 0)),
        out_shape=jax.ShapeDtypeStruct((N_NODES, d_feat), jnp.float32),
    )(part)


def _augment(W, b, width):
    d_in, d = W.shape
    Waug = jnp.pad(W, ((0, 0), (0, width - d)))
    baug = jnp.concatenate(
        [b, jnp.ones((1,), b.dtype), jnp.zeros((width - d - 1,), b.dtype)]
    ).reshape(1, width)
    return Waug, baug


def kernel(feat, edge_index_r0, edge_index_r1, edge_index_r2,
           W0_r0, b0_r0, W1_r0, b1_r0,
           W0_r1, b0_r1, W1_r1, b1_r1,
           W0_r2, b0_r2, W1_r2, b1_r2):
    edges_raw = (edge_index_r0, edge_index_r1, edge_index_r2)
    edges0 = [e.reshape(2, E_EDGES // 50, 50) for e in edges_raw]
    edges1 = [e.reshape(2, E_EDGES // 125, 125) for e in edges_raw]

    W0s = [W0_r0, W0_r1, W0_r2]
    b0s = [b.reshape(1, D_HID) for b in (b0_r0, b0_r1, b0_r2)]
    W1s, b1s = zip(*(_augment(W, b, W1_AUG)
                     for W, b in ((W1_r0, b1_r0), (W1_r1, b1_r1), (W1_r2, b1_r2))))

    feataug = jnp.concatenate(
        [feat,
         jnp.ones((N_NODES, 1), feat.dtype),
         jnp.zeros((N_NODES, W0_AUG - D_IN - 1), feat.dtype)], axis=1)
    z0 = jnp.zeros((ROWS_PER_TILE, W0_AUG), jnp.float32)
    P0 = _sc_segment_sum([feataug] * 3, edges0, z0, W0_AUG,
                         chunk=50, group=20, nbuf=4)
    T1 = _tc_combine_matmul(P0, W0s, b0s, list(W1s), list(b1s),
                            D_HID, W1_AUG)
    z1 = jnp.zeros((ROWS_PER_TILE, W1_AUG), jnp.float32)
    P1 = _sc_segment_sum(list(T1), edges1, z1, W1_AUG,
                         chunk=125, group=40, nbuf=4)
    return _tc_combine_final(P1, D_OUT)


# speedup vs baseline: 1.0386x; 1.0075x over previous
import functools

import jax
import jax.numpy as jnp
from jax import lax
from jax.experimental import pallas as pl
from jax.experimental.pallas import tpu as pltpu
from jax.experimental.pallas import tpu_sc as plsc

N_NODES = 10000
E_EDGES = 160000
D_IN, D_HID, D_OUT = 128, 128, 40
W0_AUG = 144
W1_AUG = 48

CHUNK = 100
GROUP = 10
NC, NS = 2, 16
NW = NC * NS
N_PAD = 10240
ROWS_PER_TILE = N_PAD // NS
BLK = 2000


def _sc_segment_sum(tables, edges, zrows, width, chunk, group, nbuf):
    n_chunks = E_EDGES // chunk
    cpt = n_chunks // NW
    n_groups = cpt // group
    blocks = group // nbuf
    assert cpt % group == 0 and group % nbuf == 0

    mesh = plsc.VectorSubcoreMesh(core_axis_name="c", subcore_axis_name="s")

    def body(t0, t1, t2, e0, e1, e2, zsrc, part, sidx, didx, *rest):
        rbufs = rest[0:nbuf]
        acc = rest[nbuf]
        gsems = rest[nbuf + 1:2 * nbuf + 1]
        ssems = rest[2 * nbuf + 1:3 * nbuf + 1]
        c = lax.axis_index("c")
        s = lax.axis_index("s")
        w = s * NC + c

        def wait_gather(tab, i):
            pltpu.make_async_copy(
                tab.at[pl.ds(0, chunk)], rbufs[i], gsems[i]).wait()

        def wait_scatter(i):
            pltpu.make_async_copy(
                rbufs[i], acc.at[pl.ds(0, chunk)], ssems[i]).wait()

        for r in range(3):
            tab = (t0, t1, t2)[r]
            edge = (e0, e1, e2)[r]

            pltpu.sync_copy(zsrc, acc.at[pl.ds(s * ROWS_PER_TILE,
                                               ROWS_PER_TILE)])
            plsc.subcore_barrier()

            def group_body(g, carry):
                base = w * cpt + g * group
                pltpu.sync_copy(edge.at[0, pl.ds(base, group)], sidx)
                pltpu.sync_copy(edge.at[1, pl.ds(base, group)], didx)
                for i in range(nbuf - 1):
                    @pl.when(g > 0)
                    def _(i=i):
                        wait_scatter(i)
                    pltpu.async_copy(tab.at[sidx.at[i]], rbufs[i], gsems[i])

                def block_body(blk, carry2):
                    for i in range(nbuf):
                        j = blk * nbuf + i
                        wait_gather(tab, i)
                        pltpu.async_copy(
                            rbufs[i], acc.at[didx.at[j]], ssems[i], add=True)
                        prev = (i - 1) % nbuf

                        def issue(blk=blk, i=i, prev=prev, j=j):
                            pltpu.async_copy(
                                tab.at[sidx.at[j + nbuf - 1]],
                                rbufs[prev], gsems[prev])

                        if i == 0:
                            @pl.when(jnp.logical_or(g > 0, blk > 0))
                            def _():
                                wait_scatter(prev)
                            issue()
                        else:
                            @pl.when(blk < blocks - 1)
                            def _():
                                wait_scatter(prev)
                                issue()
                    return carry2
                lax.fori_loop(0, blocks, block_body, 0)
                return carry
            lax.fori_loop(0, n_groups, group_body, 0)
            for i in range(nbuf):
                wait_scatter(i)
            plsc.subcore_barrier()

            pltpu.sync_copy(
                acc.at[pl.ds(s * ROWS_PER_TILE, ROWS_PER_TILE)],
                part.at[r, c, pl.ds(s * ROWS_PER_TILE, ROWS_PER_TILE)])
            plsc.subcore_barrier()

    fn = pl.kernel(
        body,
        out_type=jax.ShapeDtypeStruct((3, 2, N_PAD, width), jnp.float32),
        mesh=mesh,
        scratch_types=(
            [pltpu.VMEM((group, chunk), jnp.int32)] * 2
            + [pltpu.VMEM((chunk, width), jnp.float32)] * nbuf
            + [pltpu.VMEM_SHARED((N_PAD, width), jnp.float32)]
            + [pltpu.SemaphoreType.DMA] * (2 * nbuf)
        ),
        compiler_params=pltpu.CompilerParams(use_tc_tiling_on_sc=False),
    )
    return fn(*tables, *edges, zrows)


def _tc_combine_matmul(part, W0s, b0s, W1s, b1s, d_feat, width_out):
    width_in = part.shape[-1]

    def body(p_ref, w00, w01, w02, b00, b01, b02,
             w10, w11, w12, b10, b11, b12, o0, o1, o2):
        pv = p_ref[...]
        h = None
        for w0r, b0r, r in ((w00, b00, 0), (w01, b01, 1), (w02, b02, 2)):
            srel = pv[r, 0] + pv[r, 1]
            cntcol = srel[:, d_feat:d_feat + 1]
            aggf = srel[:, :d_feat] / jnp.maximum(cntcol, 1.0)
            m = (jnp.dot(aggf, w0r[...], preferred_element_type=jnp.float32)
                 + b0r[...] * (cntcol > 0.0).astype(jnp.float32))
            h = m if h is None else h + m
        h = jnp.maximum(h, 0.0)
        for wr, br, orf in ((w10, b10, o0), (w11, b11, o1), (w12, b12, o2)):
            orf[...] = (jnp.dot(h, wr[...],
                                preferred_element_type=jnp.float32)
                        + br[...])

    return pl.pallas_call(
        body,
        grid=(N_NODES // BLK,),
        in_specs=[pl.BlockSpec((3, 2, BLK, width_in), lambda i: (0, 0, i, 0))]
        + [pl.BlockSpec((d_feat, d_feat), lambda i: (0, 0))] * 3
        + [pl.BlockSpec((1, d_feat), lambda i: (0, 0))] * 3
        + [pl.BlockSpec((d_feat, width_out), lambda i: (0, 0))] * 3
        + [pl.BlockSpec((1, width_out), lambda i: (0, 0))] * 3,
        out_specs=[pl.BlockSpec((BLK, width_out), lambda i: (i, 0))] * 3,
        out_shape=[jax.ShapeDtypeStruct((N_NODES, width_out), jnp.float32)] * 3,
    )(part, *W0s, *b0s, *W1s, *b1s)


def _tc_combine_final(part, d_feat):
    width_in = part.shape[-1]

    def body(p_ref, o_ref):
        pv = p_ref[...]
        h = None
        for r in range(3):
            srel = pv[r, 0] + pv[r, 1]
            cnt = jnp.maximum(srel[:, d_feat:d_feat + 1], 1.0)
            m = srel[:, :d_feat] / cnt
            h = m if h is None else h + m
        o_ref[...] = h

    return pl.pallas_call(
        body,
        grid=(N_NODES // BLK,),
        in_specs=[pl.BlockSpec((3, 2, BLK, width_in), lambda i: (0, 0, i, 0))],
        out_specs=pl.BlockSpec((BLK, d_feat), lambda i: (i, 0)),
        out_shape=jax.ShapeDtypeStruct((N_NODES, d_feat), jnp.float32),
    )(part)


def _augment(W, b, width):
    d_in, d = W.shape
    Waug = jnp.pad(W, ((0, 0), (0, width - d)))
    baug = jnp.concatenate(
        [b, jnp.ones((1,), b.dtype), jnp.zeros((width - d - 1,), b.dtype)]
    ).reshape(1, width)
    return Waug, baug


def kernel(feat, edge_index_r0, edge_index_r1, edge_index_r2,
           W0_r0, b0_r0, W1_r0, b1_r0,
           W0_r1, b0_r1, W1_r1, b1_r1,
           W0_r2, b0_r2, W1_r2, b1_r2):
    edges_raw = (edge_index_r0, edge_index_r1, edge_index_r2)
    edges0 = [e.reshape(2, E_EDGES // 50, 50) for e in edges_raw]
    edges1 = [e.reshape(2, E_EDGES // 125, 125) for e in edges_raw]

    W0s = [W0_r0, W0_r1, W0_r2]
    b0s = [b.reshape(1, D_HID) for b in (b0_r0, b0_r1, b0_r2)]
    W1s, b1s = zip(*(_augment(W, b, W1_AUG)
                     for W, b in ((W1_r0, b1_r0), (W1_r1, b1_r1), (W1_r2, b1_r2))))

    feataug = jnp.concatenate(
        [feat,
         jnp.ones((N_NODES, 1), feat.dtype),
         jnp.zeros((N_NODES, W0_AUG - D_IN - 1), feat.dtype)], axis=1)
    z0 = jnp.zeros((ROWS_PER_TILE, W0_AUG), jnp.float32)
    P0 = _sc_segment_sum([feataug] * 3, edges0, z0, W0_AUG,
                         chunk=50, group=20, nbuf=4)
    T1 = _tc_combine_matmul(P0, W0s, b0s, list(W1s), list(b1s),
                            D_HID, W1_AUG)
    z1 = jnp.zeros((ROWS_PER_TILE, W1_AUG), jnp.float32)
    P1 = _sc_segment_sum(list(T1), edges1, z1, W1_AUG,
                         chunk=125, group=40, nbuf=4)
    return _tc_combine_final(P1, D_OUT)
